# Initial kernel scaffold; baseline (speedup 1.0000x reference)
import jax
import jax.numpy as jnp
from jax.experimental import pallas as pl


def kernel(x, edge_index1, edge_index2, label, idlabel, W1, b1, W2, b2, Wfc, bfc, prelu_a):
    raise NotImplementedError("write your pallas kernel here")


# speedup vs baseline: 4.9297x; 4.9297x over previous
import functools

import jax
import jax.numpy as jnp
from jax import lax
from jax.experimental import pallas as pl
from jax.experimental.pallas import tpu as pltpu
from jax.experimental.pallas import tpu_sc as plsc

N = 10000
D = 128
E = 320000

NC = 2
NS = 16
NW = NC * NS
EPW = E // NW
C = 80
NCHUNK = EPW // C
NP = 10240
RPT = NP // NS
DEGW = 8
ZB = 8

_MESH = plsc.VectorSubcoreMesh(core_axis_name="c", subcore_axis_name="s",
                               num_cores=NC, num_subcores=NS)


@functools.partial(
    pl.kernel,
    out_type=[
        jax.ShapeDtypeStruct((NC, NP, D), jnp.float32),
        jax.ShapeDtypeStruct((NC, NP, DEGW), jnp.float32),
    ],
    mesh=_MESH,
    scratch_types=[
        pltpu.VMEM((C,), jnp.int32),
        pltpu.VMEM((C,), jnp.int32),
        pltpu.VMEM((C, D), jnp.float32),
        pltpu.VMEM((C, DEGW), jnp.float32),
        pltpu.VMEM((ZB, D), jnp.float32),
        pltpu.VMEM((ZB, DEGW), jnp.float32),
        pltpu.SemaphoreType.DMA,
        pltpu.VMEM_SHARED((NP, D), jnp.float32),
        pltpu.VMEM_SHARED((NP, DEGW), jnp.float32),
    ],
    compiler_params=pltpu.CompilerParams(use_tc_tiling_on_sc=False),
)
def _sc_aggregate(src_hbm, dst_hbm, x_hbm, zrow_hbm, zdeg_hbm, ones_hbm,
                  pacc_hbm, pdeg_hbm,
                  srcv, dstv, rows, onesv, zbuf, zdbuf, sem, acc_sh, deg_sh):
    cid = lax.axis_index("c")
    sid = lax.axis_index("s")
    wid = cid * NS + sid

    pltpu.sync_copy(zrow_hbm, zbuf)
    pltpu.sync_copy(zdeg_hbm, zdbuf)
    pltpu.sync_copy(ones_hbm, onesv)

    @pl.loop(0, RPT // ZB)
    def _(j):
        r0 = sid * RPT + j * ZB
        pltpu.sync_copy(zbuf, acc_sh.at[pl.ds(r0, ZB)])
        pltpu.sync_copy(zdbuf, deg_sh.at[pl.ds(r0, ZB)])

    plsc.subcore_barrier()

    ebase = wid * EPW

    @pl.loop(0, NCHUNK)
    def _(k):
        off = ebase + k * C
        pltpu.sync_copy(src_hbm.at[pl.ds(off, C)], srcv)
        pltpu.sync_copy(dst_hbm.at[pl.ds(off, C)], dstv)
        pltpu.async_copy(x_hbm.at[srcv], rows, sem).wait()
        pltpu.sync_copy(rows, acc_sh.at[dstv], add=True)
        pltpu.sync_copy(onesv, deg_sh.at[dstv], add=True)

    plsc.subcore_barrier()

    @pl.loop(0, RPT // C)
    def _(j):
        r0 = sid * RPT + j * C
        pltpu.sync_copy(acc_sh.at[pl.ds(r0, C)], rows)
        pltpu.sync_copy(rows, pacc_hbm.at[cid, pl.ds(r0, C)])
        pltpu.sync_copy(deg_sh.at[pl.ds(r0, C)], onesv)
        pltpu.sync_copy(onesv, pdeg_hbm.at[cid, pl.ds(r0, C)])


RB = 400


def _tc_layer1_body(p_ref, d_ref, x_ref, w_ref, b_ref, o_ref):
    p = p_ref[0] + p_ref[1]
    deg = (jnp.sum(d_ref[0], axis=1, keepdims=True)
           + jnp.sum(d_ref[1], axis=1, keepdims=True)) * (1.0 / DEGW)
    h = (p + x_ref[...]) / (deg + 1.0)
    g = jnp.dot(h, w_ref[...], preferred_element_type=jnp.float32) + b_ref[...]
    g = jnp.maximum(g, 0.0)
    norm = jnp.maximum(jnp.sqrt(jnp.sum(g * g, axis=1, keepdims=True)), 1e-12)
    o_ref[...] = g / norm


def _tc_layer2_body(p_ref, d_ref, g_ref, w2_ref, b2_ref, wf_ref, bf_ref,
                    a_ref, o_ref):
    p = p_ref[0] + p_ref[1]
    deg = (jnp.sum(d_ref[0], axis=1, keepdims=True)
           + jnp.sum(d_ref[1], axis=1, keepdims=True)) * (1.0 / DEGW)
    h = (p + g_ref[...]) / (deg + 1.0)
    t = jnp.dot(h, w2_ref[...], preferred_element_type=jnp.float32) + b2_ref[...]
    t = jnp.maximum(t, 0.0)
    f = jnp.dot(t, wf_ref[...], preferred_element_type=jnp.float32) + bf_ref[...]
    f = jnp.where(f > 0, f, a_ref[...] * f)
    norm = jnp.maximum(jnp.sqrt(jnp.sum(f * f, axis=1, keepdims=True)), 1e-12)
    o_ref[...] = f / norm


def _tc_layer1(p, d, x, W, b):
    return pl.pallas_call(
        _tc_layer1_body,
        grid=(N // RB,),
        in_specs=[
            pl.BlockSpec((NC, RB, D), lambda i: (0, i, 0)),
            pl.BlockSpec((NC, RB, DEGW), lambda i: (0, i, 0)),
            pl.BlockSpec((RB, D), lambda i: (i, 0)),
            pl.BlockSpec((D, D), lambda i: (0, 0)),
            pl.BlockSpec((1, D), lambda i: (0, 0)),
        ],
        out_specs=pl.BlockSpec((RB, D), lambda i: (i, 0)),
        out_shape=jax.ShapeDtypeStruct((N, D), jnp.float32),
    )(p, d, x, W, b)


def _tc_layer2(p, d, g, W2, b2, Wf, bf, a):
    return pl.pallas_call(
        _tc_layer2_body,
        grid=(N // RB,),
        in_specs=[
            pl.BlockSpec((NC, RB, D), lambda i: (0, i, 0)),
            pl.BlockSpec((NC, RB, DEGW), lambda i: (0, i, 0)),
            pl.BlockSpec((RB, D), lambda i: (i, 0)),
            pl.BlockSpec((D, D), lambda i: (0, 0)),
            pl.BlockSpec((1, D), lambda i: (0, 0)),
            pl.BlockSpec((D, D), lambda i: (0, 0)),
            pl.BlockSpec((1, D), lambda i: (0, 0)),
            pl.BlockSpec((1, D), lambda i: (0, 0)),
        ],
        out_specs=pl.BlockSpec((RB, D), lambda i: (i, 0)),
        out_shape=jax.ShapeDtypeStruct((N, D), jnp.float32),
    )(p, d, g, W2, b2, Wf, bf, a)


def kernel(x, edge_index1, edge_index2, label, idlabel,
           W1, b1, W2, b2, Wfc, bfc, prelu_a):
    del label, idlabel
    zrow = jnp.zeros((ZB, D), jnp.float32)
    zdeg = jnp.zeros((ZB, DEGW), jnp.float32)
    ones = jnp.ones((C, DEGW), jnp.float32)

    p1, d1 = _sc_aggregate(edge_index1[0], edge_index1[1], x, zrow, zdeg, ones)
    g1 = _tc_layer1(p1, d1, x, W1, b1.reshape(1, D))
    p2, d2 = _sc_aggregate(edge_index2[0], edge_index2[1], g1, zrow, zdeg, ones)
    out = _tc_layer2(p2, d2, g1, W2, b2.reshape(1, D), Wfc, bfc.reshape(1, D),
                     prelu_a.reshape(1, D))
    return out


# speedup vs baseline: 8.5161x; 1.7275x over previous
import functools

import jax
import jax.numpy as jnp
from jax import lax
from jax.experimental import pallas as pl
from jax.experimental.pallas import tpu as pltpu
from jax.experimental.pallas import tpu_sc as plsc

N = 10000
D = 128
E = 320000

NC = 2
NS = 16
NW = NC * NS
EPW = E // NW
C = 80
NCHUNK = EPW // C
NP = 10240
RPT = NP // NS
DEGW = 8
ZB = 8

_MESH = plsc.VectorSubcoreMesh(core_axis_name="c", subcore_axis_name="s",
                               num_cores=NC, num_subcores=NS)


@functools.partial(
    pl.kernel,
    out_type=[
        jax.ShapeDtypeStruct((NC, NP, D), jnp.float32),
        jax.ShapeDtypeStruct((NC, NP, DEGW), jnp.float32),
    ],
    mesh=_MESH,
    scratch_types=[
        pltpu.VMEM((NCHUNK, C), jnp.int32),
        pltpu.VMEM((NCHUNK, C), jnp.int32),
        pltpu.VMEM((C, D), jnp.float32),
        pltpu.VMEM((C, D), jnp.float32),
        pltpu.VMEM((C, DEGW), jnp.float32),
        pltpu.VMEM((ZB, D), jnp.float32),
        pltpu.VMEM((ZB, DEGW), jnp.float32),
        pltpu.SemaphoreType.DMA,
        pltpu.SemaphoreType.DMA,
        pltpu.SemaphoreType.DMA,
        pltpu.VMEM_SHARED((NP, D), jnp.float32),
        pltpu.VMEM_SHARED((NP, DEGW), jnp.float32),
    ],
    compiler_params=pltpu.CompilerParams(use_tc_tiling_on_sc=False),
)
def _sc_aggregate(src_hbm, dst_hbm, x_hbm, zrow_hbm, zdeg_hbm, ones_hbm,
                  pacc_hbm, pdeg_hbm,
                  srcb, dstb, rows0, rows1, onesv, zbuf, zdbuf,
                  sem0, sem1, semd, acc_sh, deg_sh):
    cid = lax.axis_index("c")
    sid = lax.axis_index("s")
    wid = cid * NS + sid

    cbase = wid * NCHUNK
    dsrc = pltpu.async_copy(src_hbm.at[pl.ds(cbase, NCHUNK)], srcb, sem0)
    ddst = pltpu.async_copy(dst_hbm.at[pl.ds(cbase, NCHUNK)], dstb, sem1)

    pltpu.sync_copy(zrow_hbm, zbuf)
    pltpu.sync_copy(zdeg_hbm, zdbuf)
    pltpu.sync_copy(ones_hbm, onesv)

    @pl.loop(0, RPT // ZB)
    def _(j):
        r0 = sid * RPT + j * ZB
        pltpu.sync_copy(zbuf, acc_sh.at[pl.ds(r0, ZB)])
        pltpu.sync_copy(zdbuf, deg_sh.at[pl.ds(r0, ZB)])

    dsrc.wait()
    ddst.wait()
    plsc.subcore_barrier()

    @pl.loop(0, NCHUNK // 2)
    def _(j):
        e = 2 * j
        o = 2 * j + 1
        ge = pltpu.async_copy(x_hbm.at[srcb.at[e]], rows0, sem0)
        go = pltpu.async_copy(x_hbm.at[srcb.at[o]], rows1, sem1)
        de = pltpu.async_copy(onesv, deg_sh.at[dstb.at[e]], semd, add=True)
        do = pltpu.async_copy(onesv, deg_sh.at[dstb.at[o]], semd, add=True)
        ge.wait()
        pltpu.sync_copy(rows0, acc_sh.at[dstb.at[e]], add=True)
        go.wait()
        pltpu.sync_copy(rows1, acc_sh.at[dstb.at[o]], add=True)
        de.wait()
        do.wait()

    last = NCHUNK - 1
    pltpu.async_copy(x_hbm.at[srcb.at[last]], rows0, sem0).wait()
    pltpu.sync_copy(rows0, acc_sh.at[dstb.at[last]], add=True)
    pltpu.sync_copy(onesv, deg_sh.at[dstb.at[last]], add=True)

    plsc.subcore_barrier()

    @pl.loop(0, RPT // C)
    def _(j):
        r0 = sid * RPT + j * C
        pltpu.sync_copy(acc_sh.at[pl.ds(r0, C)], rows0)
        pltpu.sync_copy(rows0, pacc_hbm.at[cid, pl.ds(r0, C)])
        pltpu.sync_copy(deg_sh.at[pl.ds(r0, C)], onesv)
        pltpu.sync_copy(onesv, pdeg_hbm.at[cid, pl.ds(r0, C)])


RB = 400


def _tc_layer1_body(p_ref, d_ref, x_ref, w_ref, b_ref, o_ref):
    p = p_ref[0] + p_ref[1]
    deg = (jnp.sum(d_ref[0], axis=1, keepdims=True)
           + jnp.sum(d_ref[1], axis=1, keepdims=True)) * (1.0 / DEGW)
    h = (p + x_ref[...]) / (deg + 1.0)
    g = jnp.dot(h, w_ref[...], preferred_element_type=jnp.float32) + b_ref[...]
    g = jnp.maximum(g, 0.0)
    norm = jnp.maximum(jnp.sqrt(jnp.sum(g * g, axis=1, keepdims=True)), 1e-12)
    o_ref[...] = g / norm


def _tc_layer2_body(p_ref, d_ref, g_ref, w2_ref, b2_ref, wf_ref, bf_ref,
                    a_ref, o_ref):
    p = p_ref[0] + p_ref[1]
    deg = (jnp.sum(d_ref[0], axis=1, keepdims=True)
           + jnp.sum(d_ref[1], axis=1, keepdims=True)) * (1.0 / DEGW)
    h = (p + g_ref[...]) / (deg + 1.0)
    t = jnp.dot(h, w2_ref[...], preferred_element_type=jnp.float32) + b2_ref[...]
    t = jnp.maximum(t, 0.0)
    f = jnp.dot(t, wf_ref[...], preferred_element_type=jnp.float32) + bf_ref[...]
    f = jnp.where(f > 0, f, a_ref[...] * f)
    norm = jnp.maximum(jnp.sqrt(jnp.sum(f * f, axis=1, keepdims=True)), 1e-12)
    o_ref[...] = f / norm


def _tc_layer1(p, d, x, W, b):
    return pl.pallas_call(
        _tc_layer1_body,
        grid=(N // RB,),
        in_specs=[
            pl.BlockSpec((NC, RB, D), lambda i: (0, i, 0)),
            pl.BlockSpec((NC, RB, DEGW), lambda i: (0, i, 0)),
            pl.BlockSpec((RB, D), lambda i: (i, 0)),
            pl.BlockSpec((D, D), lambda i: (0, 0)),
            pl.BlockSpec((1, D), lambda i: (0, 0)),
        ],
        out_specs=pl.BlockSpec((RB, D), lambda i: (i, 0)),
        out_shape=jax.ShapeDtypeStruct((N, D), jnp.float32),
    )(p, d, x, W, b)


def _tc_layer2(p, d, g, W2, b2, Wf, bf, a):
    return pl.pallas_call(
        _tc_layer2_body,
        grid=(N // RB,),
        in_specs=[
            pl.BlockSpec((NC, RB, D), lambda i: (0, i, 0)),
            pl.BlockSpec((NC, RB, DEGW), lambda i: (0, i, 0)),
            pl.BlockSpec((RB, D), lambda i: (i, 0)),
            pl.BlockSpec((D, D), lambda i: (0, 0)),
            pl.BlockSpec((1, D), lambda i: (0, 0)),
            pl.BlockSpec((D, D), lambda i: (0, 0)),
            pl.BlockSpec((1, D), lambda i: (0, 0)),
            pl.BlockSpec((1, D), lambda i: (0, 0)),
        ],
        out_specs=pl.BlockSpec((RB, D), lambda i: (i, 0)),
        out_shape=jax.ShapeDtypeStruct((N, D), jnp.float32),
    )(p, d, g, W2, b2, Wf, bf, a)


def kernel(x, edge_index1, edge_index2, label, idlabel,
           W1, b1, W2, b2, Wfc, bfc, prelu_a):
    del label, idlabel
    zrow = jnp.zeros((ZB, D), jnp.float32)
    zdeg = jnp.zeros((ZB, DEGW), jnp.float32)
    ones = jnp.ones((C, DEGW), jnp.float32)

    s1 = edge_index1[0].reshape(E // C, C)
    t1 = edge_index1[1].reshape(E // C, C)
    s2 = edge_index2[0].reshape(E // C, C)
    t2 = edge_index2[1].reshape(E // C, C)
    p1, d1 = _sc_aggregate(s1, t1, x, zrow, zdeg, ones)
    g1 = _tc_layer1(p1, d1, x, W1, b1.reshape(1, D))
    p2, d2 = _sc_aggregate(s2, t2, g1, zrow, zdeg, ones)
    out = _tc_layer2(p2, d2, g1, W2, b2.reshape(1, D), Wfc, bfc.reshape(1, D),
                     prelu_a.reshape(1, D))
    return out


# speedup vs baseline: 9.9191x; 1.1648x over previous
import functools

import jax
import jax.numpy as jnp
from jax import lax
from jax.experimental import pallas as pl
from jax.experimental.pallas import tpu as pltpu
from jax.experimental.pallas import tpu_sc as plsc

N = 10000
D = 128
E = 320000

NC = 2
NS = 16
NW = NC * NS
EPW = E // NW
C = 80
NCHUNK = EPW // C
NP = 10240
RPT = NP // NS
DEGW = 8

_MESH = plsc.VectorSubcoreMesh(core_axis_name="c", subcore_axis_name="s",
                               num_cores=NC, num_subcores=NS)


@functools.partial(
    pl.kernel,
    out_type=[
        jax.ShapeDtypeStruct((NC, NP, D), jnp.float32),
        jax.ShapeDtypeStruct((NC, NP, DEGW), jnp.float32),
    ],
    mesh=_MESH,
    scratch_types=[
        pltpu.VMEM((NCHUNK, C), jnp.int32),
        pltpu.VMEM((NCHUNK, C), jnp.int32),
        pltpu.VMEM((C, D), jnp.float32),
        pltpu.VMEM((C, D), jnp.float32),
        pltpu.VMEM((C, DEGW), jnp.float32),
        pltpu.SemaphoreType.DMA,
        pltpu.SemaphoreType.DMA,
        pltpu.SemaphoreType.DMA,
        pltpu.SemaphoreType.DMA,
        pltpu.SemaphoreType.DMA,
        pltpu.VMEM_SHARED((NP, D), jnp.float32),
        pltpu.VMEM_SHARED((NP, DEGW), jnp.float32),
    ],
    compiler_params=pltpu.CompilerParams(use_tc_tiling_on_sc=False),
)
def _sc_aggregate(src_hbm, dst_hbm, x_hbm, zrow_hbm, zdeg_hbm, ones_hbm,
                  pacc_hbm, pdeg_hbm,
                  srcb, dstb, rows0, rows1, onesv,
                  sg0, sg1, ss0, ss1, semd, acc_sh, deg_sh):
    cid = lax.axis_index("c")
    sid = lax.axis_index("s")
    wid = cid * NS + sid
    band = pl.ds(sid * RPT, RPT)

    cbase = wid * NCHUNK
    dsrc = pltpu.async_copy(src_hbm.at[pl.ds(cbase, NCHUNK)], srcb, sg0)
    ddst = pltpu.async_copy(dst_hbm.at[pl.ds(cbase, NCHUNK)], dstb, sg1)
    dz0 = pltpu.async_copy(zrow_hbm, acc_sh.at[band], ss0)
    dz1 = pltpu.async_copy(zdeg_hbm, deg_sh.at[band], ss1)
    pltpu.sync_copy(ones_hbm, onesv)
    dsrc.wait()
    ddst.wait()
    dz0.wait()
    dz1.wait()
    plsc.subcore_barrier()

    @pl.loop(0, NCHUNK // 2)
    def _(j):
        e = 2 * j
        o = 2 * j + 1

        @pl.when(j > 0)
        def _():
            pltpu.make_async_copy(rows0, acc_sh.at[dstb.at[e]], ss0).wait()
            pltpu.make_async_copy(onesv, deg_sh.at[dstb.at[e]], semd).wait()

        ge = pltpu.async_copy(x_hbm.at[srcb.at[e]], rows0, sg0)

        @pl.when(j > 0)
        def _():
            pltpu.make_async_copy(rows1, acc_sh.at[dstb.at[o]], ss1).wait()
            pltpu.make_async_copy(onesv, deg_sh.at[dstb.at[o]], semd).wait()

        go = pltpu.async_copy(x_hbm.at[srcb.at[o]], rows1, sg1)
        ge.wait()
        pltpu.async_copy(rows0, acc_sh.at[dstb.at[e]], ss0, add=True)
        pltpu.async_copy(onesv, deg_sh.at[dstb.at[e]], semd, add=True)
        go.wait()
        pltpu.async_copy(rows1, acc_sh.at[dstb.at[o]], ss1, add=True)
        pltpu.async_copy(onesv, deg_sh.at[dstb.at[o]], semd, add=True)

    last = NCHUNK - 1
    pltpu.make_async_copy(rows0, acc_sh.at[dstb.at[last]], ss0).wait()
    pltpu.make_async_copy(rows1, acc_sh.at[dstb.at[last]], ss1).wait()
    pltpu.make_async_copy(onesv, deg_sh.at[dstb.at[last]], semd).wait()
    pltpu.make_async_copy(onesv, deg_sh.at[dstb.at[last]], semd).wait()
    pltpu.async_copy(x_hbm.at[srcb.at[last]], rows0, sg0).wait()
    pltpu.sync_copy(rows0, acc_sh.at[dstb.at[last]], add=True)
    pltpu.sync_copy(onesv, deg_sh.at[dstb.at[last]], add=True)

    plsc.subcore_barrier()

    dw0 = pltpu.async_copy(acc_sh.at[band], pacc_hbm.at[cid, band], ss0)
    dw1 = pltpu.async_copy(deg_sh.at[band], pdeg_hbm.at[cid, band], ss1)
    dw0.wait()
    dw1.wait()


RB = 400


def _tc_layer1_body(p_ref, d_ref, x_ref, w_ref, b_ref, o_ref):
    p = p_ref[0] + p_ref[1]
    deg = (jnp.sum(d_ref[0], axis=1, keepdims=True)
           + jnp.sum(d_ref[1], axis=1, keepdims=True)) * (1.0 / DEGW)
    h = (p + x_ref[...]) / (deg + 1.0)
    g = jnp.dot(h, w_ref[...], preferred_element_type=jnp.float32) + b_ref[...]
    g = jnp.maximum(g, 0.0)
    norm = jnp.maximum(jnp.sqrt(jnp.sum(g * g, axis=1, keepdims=True)), 1e-12)
    o_ref[...] = g / norm


def _tc_layer2_body(p_ref, d_ref, g_ref, w2_ref, b2_ref, wf_ref, bf_ref,
                    a_ref, o_ref):
    p = p_ref[0] + p_ref[1]
    deg = (jnp.sum(d_ref[0], axis=1, keepdims=True)
           + jnp.sum(d_ref[1], axis=1, keepdims=True)) * (1.0 / DEGW)
    h = (p + g_ref[...]) / (deg + 1.0)
    t = jnp.dot(h, w2_ref[...], preferred_element_type=jnp.float32) + b2_ref[...]
    t = jnp.maximum(t, 0.0)
    f = jnp.dot(t, wf_ref[...], preferred_element_type=jnp.float32) + bf_ref[...]
    f = jnp.where(f > 0, f, a_ref[...] * f)
    norm = jnp.maximum(jnp.sqrt(jnp.sum(f * f, axis=1, keepdims=True)), 1e-12)
    o_ref[...] = f / norm


def _tc_layer1(p, d, x, W, b):
    return pl.pallas_call(
        _tc_layer1_body,
        grid=(N // RB,),
        in_specs=[
            pl.BlockSpec((NC, RB, D), lambda i: (0, i, 0)),
            pl.BlockSpec((NC, RB, DEGW), lambda i: (0, i, 0)),
            pl.BlockSpec((RB, D), lambda i: (i, 0)),
            pl.BlockSpec((D, D), lambda i: (0, 0)),
            pl.BlockSpec((1, D), lambda i: (0, 0)),
        ],
        out_specs=pl.BlockSpec((RB, D), lambda i: (i, 0)),
        out_shape=jax.ShapeDtypeStruct((N, D), jnp.float32),
    )(p, d, x, W, b)


def _tc_layer2(p, d, g, W2, b2, Wf, bf, a):
    return pl.pallas_call(
        _tc_layer2_body,
        grid=(N // RB,),
        in_specs=[
            pl.BlockSpec((NC, RB, D), lambda i: (0, i, 0)),
            pl.BlockSpec((NC, RB, DEGW), lambda i: (0, i, 0)),
            pl.BlockSpec((RB, D), lambda i: (i, 0)),
            pl.BlockSpec((D, D), lambda i: (0, 0)),
            pl.BlockSpec((1, D), lambda i: (0, 0)),
            pl.BlockSpec((D, D), lambda i: (0, 0)),
            pl.BlockSpec((1, D), lambda i: (0, 0)),
            pl.BlockSpec((1, D), lambda i: (0, 0)),
        ],
        out_specs=pl.BlockSpec((RB, D), lambda i: (i, 0)),
        out_shape=jax.ShapeDtypeStruct((N, D), jnp.float32),
    )(p, d, g, W2, b2, Wf, bf, a)


def kernel(x, edge_index1, edge_index2, label, idlabel,
           W1, b1, W2, b2, Wfc, bfc, prelu_a):
    del label, idlabel
    zrow = jnp.zeros((RPT, D), jnp.float32)
    zdeg = jnp.zeros((RPT, DEGW), jnp.float32)
    ones = jnp.ones((C, DEGW), jnp.float32)

    s1 = edge_index1[0].reshape(E // C, C)
    t1 = edge_index1[1].reshape(E // C, C)
    s2 = edge_index2[0].reshape(E // C, C)
    t2 = edge_index2[1].reshape(E // C, C)
    p1, d1 = _sc_aggregate(s1, t1, x, zrow, zdeg, ones)
    g1 = _tc_layer1(p1, d1, x, W1, b1.reshape(1, D))
    p2, d2 = _sc_aggregate(s2, t2, g1, zrow, zdeg, ones)
    out = _tc_layer2(p2, d2, g1, W2, b2.reshape(1, D), Wfc, bfc.reshape(1, D),
                     prelu_a.reshape(1, D))
    return out


# speedup vs baseline: 10.7539x; 1.0842x over previous
import functools

import jax
import jax.numpy as jnp
from jax import lax
from jax.experimental import pallas as pl
from jax.experimental.pallas import tpu as pltpu
from jax.experimental.pallas import tpu_sc as plsc

N = 10000
D = 128
E = 320000

NC = 2
NS = 16
NW = NC * NS
EPW = E // NW
C = 80
NCHUNK = EPW // C
NP = 10240
RPT = NP // NS
DEGW = 8

_MESH = plsc.VectorSubcoreMesh(core_axis_name="c", subcore_axis_name="s",
                               num_cores=NC, num_subcores=NS)


@functools.partial(
    pl.kernel,
    out_type=[
        jax.ShapeDtypeStruct((NC, NP, D), jnp.float32),
        jax.ShapeDtypeStruct((NC, NP, DEGW), jnp.float32),
    ],
    mesh=_MESH,
    scratch_types=[
        pltpu.VMEM((NCHUNK, C), jnp.int32),
        pltpu.VMEM((NCHUNK, C), jnp.int32),
        pltpu.VMEM((C, D), jnp.float32),
        pltpu.VMEM((C, D), jnp.float32),
        pltpu.VMEM((C, DEGW), jnp.float32),
        pltpu.SemaphoreType.DMA,
        pltpu.SemaphoreType.DMA,
        pltpu.SemaphoreType.DMA,
        pltpu.SemaphoreType.DMA,
        pltpu.SemaphoreType.DMA,
        pltpu.VMEM_SHARED((NP, D), jnp.float32),
        pltpu.VMEM_SHARED((NP, DEGW), jnp.float32),
    ],
    compiler_params=pltpu.CompilerParams(use_tc_tiling_on_sc=False),
)
def _sc_aggregate(ei_hbm, x_hbm, zrow_hbm, zdeg_hbm, ones_hbm,
                  pacc_hbm, pdeg_hbm,
                  srcb, dstb, rows0, rows1, onesv,
                  sg0, sg1, ss0, ss1, semd, acc_sh, deg_sh):
    cid = lax.axis_index("c")
    sid = lax.axis_index("s")
    wid = cid * NS + sid
    band = pl.ds(sid * RPT, RPT)

    cbase = wid * NCHUNK
    dsrc = pltpu.async_copy(ei_hbm.at[0, pl.ds(cbase, NCHUNK)], srcb, sg0)
    ddst = pltpu.async_copy(ei_hbm.at[1, pl.ds(cbase, NCHUNK)], dstb, sg1)
    dz0 = pltpu.async_copy(zrow_hbm, acc_sh.at[band], ss0)
    dz1 = pltpu.async_copy(zdeg_hbm, deg_sh.at[band], ss1)
    pltpu.sync_copy(ones_hbm, onesv)
    dsrc.wait()
    ddst.wait()
    dz0.wait()
    dz1.wait()
    plsc.subcore_barrier()

    @pl.loop(0, NCHUNK // 2)
    def _(j):
        e = 2 * j
        o = 2 * j + 1

        @pl.when(j > 0)
        def _():
            pltpu.make_async_copy(rows0, acc_sh.at[dstb.at[e]], ss0).wait()
            pltpu.make_async_copy(onesv, deg_sh.at[dstb.at[e]], semd).wait()

        ge = pltpu.async_copy(x_hbm.at[srcb.at[e]], rows0, sg0)

        @pl.when(j > 0)
        def _():
            pltpu.make_async_copy(rows1, acc_sh.at[dstb.at[o]], ss1).wait()
            pltpu.make_async_copy(onesv, deg_sh.at[dstb.at[o]], semd).wait()

        go = pltpu.async_copy(x_hbm.at[srcb.at[o]], rows1, sg1)
        ge.wait()
        pltpu.async_copy(rows0, acc_sh.at[dstb.at[e]], ss0, add=True)
        pltpu.async_copy(onesv, deg_sh.at[dstb.at[e]], semd, add=True)
        go.wait()
        pltpu.async_copy(rows1, acc_sh.at[dstb.at[o]], ss1, add=True)
        pltpu.async_copy(onesv, deg_sh.at[dstb.at[o]], semd, add=True)

    last = NCHUNK - 1
    pltpu.make_async_copy(rows0, acc_sh.at[dstb.at[last]], ss0).wait()
    pltpu.make_async_copy(rows1, acc_sh.at[dstb.at[last]], ss1).wait()
    pltpu.make_async_copy(onesv, deg_sh.at[dstb.at[last]], semd).wait()
    pltpu.make_async_copy(onesv, deg_sh.at[dstb.at[last]], semd).wait()
    pltpu.async_copy(x_hbm.at[srcb.at[last]], rows0, sg0).wait()
    pltpu.sync_copy(rows0, acc_sh.at[dstb.at[last]], add=True)
    pltpu.sync_copy(onesv, deg_sh.at[dstb.at[last]], add=True)

    plsc.subcore_barrier()

    dw0 = pltpu.async_copy(acc_sh.at[band], pacc_hbm.at[cid, band], ss0)
    dw1 = pltpu.async_copy(deg_sh.at[band], pdeg_hbm.at[cid, band], ss1)
    dw0.wait()
    dw1.wait()


RB = 1000


def _tc_layer1_body(p_ref, d_ref, x_ref, w_ref, b_ref, o_ref):
    p = p_ref[0] + p_ref[1]
    deg = (jnp.sum(d_ref[0], axis=1, keepdims=True)
           + jnp.sum(d_ref[1], axis=1, keepdims=True)) * (1.0 / DEGW)
    h = (p + x_ref[...]) / (deg + 1.0)
    g = jnp.dot(h, w_ref[...], preferred_element_type=jnp.float32) + b_ref[...]
    g = jnp.maximum(g, 0.0)
    norm = jnp.maximum(jnp.sqrt(jnp.sum(g * g, axis=1, keepdims=True)), 1e-12)
    o_ref[...] = g / norm


def _tc_layer2_body(p_ref, d_ref, g_ref, w2_ref, b2_ref, wf_ref, bf_ref,
                    a_ref, o_ref):
    p = p_ref[0] + p_ref[1]
    deg = (jnp.sum(d_ref[0], axis=1, keepdims=True)
           + jnp.sum(d_ref[1], axis=1, keepdims=True)) * (1.0 / DEGW)
    h = (p + g_ref[...]) / (deg + 1.0)
    t = jnp.dot(h, w2_ref[...], preferred_element_type=jnp.float32) + b2_ref[...]
    t = jnp.maximum(t, 0.0)
    f = jnp.dot(t, wf_ref[...], preferred_element_type=jnp.float32) + bf_ref[...]
    f = jnp.where(f > 0, f, a_ref[...] * f)
    norm = jnp.maximum(jnp.sqrt(jnp.sum(f * f, axis=1, keepdims=True)), 1e-12)
    o_ref[...] = f / norm


def _tc_layer1(p, d, x, W, b):
    return pl.pallas_call(
        _tc_layer1_body,
        grid=(N // RB,),
        in_specs=[
            pl.BlockSpec((NC, RB, D), lambda i: (0, i, 0)),
            pl.BlockSpec((NC, RB, DEGW), lambda i: (0, i, 0)),
            pl.BlockSpec((RB, D), lambda i: (i, 0)),
            pl.BlockSpec((D, D), lambda i: (0, 0)),
            pl.BlockSpec((1, D), lambda i: (0, 0)),
        ],
        out_specs=pl.BlockSpec((RB, D), lambda i: (i, 0)),
        out_shape=jax.ShapeDtypeStruct((N, D), jnp.float32),
    )(p, d, x, W, b)


def _tc_layer2(p, d, g, W2, b2, Wf, bf, a):
    return pl.pallas_call(
        _tc_layer2_body,
        grid=(N // RB,),
        in_specs=[
            pl.BlockSpec((NC, RB, D), lambda i: (0, i, 0)),
            pl.BlockSpec((NC, RB, DEGW), lambda i: (0, i, 0)),
            pl.BlockSpec((RB, D), lambda i: (i, 0)),
            pl.BlockSpec((D, D), lambda i: (0, 0)),
            pl.BlockSpec((1, D), lambda i: (0, 0)),
            pl.BlockSpec((D, D), lambda i: (0, 0)),
            pl.BlockSpec((1, D), lambda i: (0, 0)),
            pl.BlockSpec((1, D), lambda i: (0, 0)),
        ],
        out_specs=pl.BlockSpec((RB, D), lambda i: (i, 0)),
        out_shape=jax.ShapeDtypeStruct((N, D), jnp.float32),
    )(p, d, g, W2, b2, Wf, bf, a)


def kernel(x, edge_index1, edge_index2, label, idlabel,
           W1, b1, W2, b2, Wfc, bfc, prelu_a):
    del label, idlabel
    zrow = jnp.zeros((RPT, D), jnp.float32)
    zdeg = jnp.zeros((RPT, DEGW), jnp.float32)
    ones = jnp.ones((C, DEGW), jnp.float32)

    ei1 = edge_index1.reshape(2, E // C, C)
    ei2 = edge_index2.reshape(2, E // C, C)
    p1, d1 = _sc_aggregate(ei1, x, zrow, zdeg, ones)
    g1 = _tc_layer1(p1, d1, x, W1, b1.reshape(1, D))
    p2, d2 = _sc_aggregate(ei2, g1, zrow, zdeg, ones)
    out = _tc_layer2(p2, d2, g1, W2, b2.reshape(1, D), Wfc, bfc.reshape(1, D),
                     prelu_a.reshape(1, D))
    return out


# speedup vs baseline: 11.6636x; 1.0846x over previous
import functools

import jax
import jax.numpy as jnp
from jax import lax
from jax.experimental import pallas as pl
from jax.experimental.pallas import tpu as pltpu
from jax.experimental.pallas import tpu_sc as plsc

N = 10000
D = 128
E = 320000

NC = 2
NS = 16
NW = NC * NS
EPW = E // NW
C = 80
NCHUNK = EPW // C
NP = 10240
RPT = NP // NS
DEGW = 8

_MESH = plsc.VectorSubcoreMesh(core_axis_name="c", subcore_axis_name="s",
                               num_cores=NC, num_subcores=NS)


@functools.partial(
    pl.kernel,
    out_type=[
        jax.ShapeDtypeStruct((NC, NP, D), jnp.float32),
        jax.ShapeDtypeStruct((NC, NP, DEGW), jnp.float32),
    ],
    mesh=_MESH,
    scratch_types=[
        pltpu.VMEM((NCHUNK, C), jnp.int32),
        pltpu.VMEM((C, D), jnp.float32),
        pltpu.VMEM((C, D), jnp.float32),
        pltpu.VMEM((C, D), jnp.float32),
        pltpu.VMEM((C,), jnp.int32),
        pltpu.VMEM((C,), jnp.int32),
        pltpu.VMEM((C,), jnp.int32),
        pltpu.VMEM((C,), jnp.int32),
        pltpu.VMEM((C,), jnp.int32),
        pltpu.VMEM((C,), jnp.int32),
        pltpu.VMEM((C, DEGW), jnp.float32),
        pltpu.SemaphoreType.DMA,
        pltpu.SemaphoreType.DMA,
        pltpu.SemaphoreType.DMA,
        pltpu.SemaphoreType.DMA,
        pltpu.SemaphoreType.DMA,
        pltpu.SemaphoreType.DMA,
        pltpu.SemaphoreType.DMA,
        pltpu.VMEM_SHARED((NP, D), jnp.float32),
        pltpu.VMEM_SHARED((NP, DEGW), jnp.float32),
    ],
    compiler_params=pltpu.CompilerParams(use_tc_tiling_on_sc=False),
)
def _sc_aggregate(ei_hbm, x_hbm, zrow_hbm, zdeg_hbm, ones_hbm,
                  pacc_hbm, pdeg_hbm,
                  eipb, rows0, rows1, rows2, sv0, sv1, sv2, dv0, dv1, dv2,
                  onesv, sg0, sg1, sg2, ss0, ss1, ss2, semd, acc_sh, deg_sh):
    cid = lax.axis_index("c")
    sid = lax.axis_index("s")
    wid = cid * NS + sid
    band = pl.ds(sid * RPT, RPT)
    rows = (rows0, rows1, rows2)
    sv = (sv0, sv1, sv2)
    dv = (dv0, dv1, dv2)
    sg = (sg0, sg1, sg2)
    ss = (ss0, ss1, ss2)

    cbase = wid * NCHUNK
    dei = pltpu.async_copy(ei_hbm.at[pl.ds(cbase, NCHUNK)], eipb, sg0)
    dz0 = pltpu.async_copy(zrow_hbm, acc_sh.at[band], ss0)
    dz1 = pltpu.async_copy(zdeg_hbm, deg_sh.at[band], ss1)
    pltpu.sync_copy(ones_hbm, onesv)
    dei.wait()
    dz0.wait()
    dz1.wait()
    plsc.subcore_barrier()

    def unpack(k, b):
        for i in range(C // 16):
            w = eipb[k, pl.ds(i * 16, 16)]
            sv[b][pl.ds(i * 16, 16)] = w & 0xFFFF
            dv[b][pl.ds(i * 16, 16)] = lax.shift_right_logical(w, 16)

    def drain(b):
        pltpu.make_async_copy(rows[b], acc_sh.at[dv[b]], ss[b]).wait()
        pltpu.make_async_copy(onesv, deg_sh.at[dv[b]], semd).wait()

    def gather(k, b):
        pltpu.async_copy(x_hbm.at[sv[b]], rows[b], sg[b])

    def gwait(b):
        pltpu.make_async_copy(x_hbm.at[sv[b]], rows[b], sg[b]).wait()

    def scatter(b):
        pltpu.async_copy(rows[b], acc_sh.at[dv[b]], ss[b], add=True)
        pltpu.async_copy(onesv, deg_sh.at[dv[b]], semd, add=True)

    @pl.loop(0, NCHUNK // 3)
    def _(j):
        k0 = 3 * j
        for b in range(3):
            @pl.when(j > 0)
            def _(b=b):
                drain(b)
            unpack(k0 + b, b)
            gather(k0 + b, b)
        for b in range(3):
            gwait(b)
            scatter(b)

    t0 = 3 * (NCHUNK // 3)
    for b in range(3):
        drain(b)
    for i, k in enumerate(range(t0, NCHUNK)):
        unpack(k, i)
        gather(k, i)
    for i in range(NCHUNK - t0):
        gwait(i)
        scatter(i)
    for i in range(NCHUNK - t0):
        drain(i)

    plsc.subcore_barrier()

    dw0 = pltpu.async_copy(acc_sh.at[band], pacc_hbm.at[cid, band], ss0)
    dw1 = pltpu.async_copy(deg_sh.at[band], pdeg_hbm.at[cid, band], ss1)
    dw0.wait()
    dw1.wait()


RB = 1000


def _tc_layer1_body(p_ref, d_ref, x_ref, w_ref, b_ref, o_ref):
    p = p_ref[0] + p_ref[1]
    deg = (jnp.sum(d_ref[0], axis=1, keepdims=True)
           + jnp.sum(d_ref[1], axis=1, keepdims=True)) * (1.0 / DEGW)
    h = (p + x_ref[...]) / (deg + 1.0)
    g = jnp.dot(h, w_ref[...], preferred_element_type=jnp.float32) + b_ref[...]
    g = jnp.maximum(g, 0.0)
    norm = jnp.maximum(jnp.sqrt(jnp.sum(g * g, axis=1, keepdims=True)), 1e-12)
    o_ref[...] = g / norm


def _tc_layer2_body(p_ref, d_ref, g_ref, w2_ref, b2_ref, wf_ref, bf_ref,
                    a_ref, o_ref):
    p = p_ref[0] + p_ref[1]
    deg = (jnp.sum(d_ref[0], axis=1, keepdims=True)
           + jnp.sum(d_ref[1], axis=1, keepdims=True)) * (1.0 / DEGW)
    h = (p + g_ref[...]) / (deg + 1.0)
    t = jnp.dot(h, w2_ref[...], preferred_element_type=jnp.float32) + b2_ref[...]
    t = jnp.maximum(t, 0.0)
    f = jnp.dot(t, wf_ref[...], preferred_element_type=jnp.float32) + bf_ref[...]
    f = jnp.where(f > 0, f, a_ref[...] * f)
    norm = jnp.maximum(jnp.sqrt(jnp.sum(f * f, axis=1, keepdims=True)), 1e-12)
    o_ref[...] = f / norm


def _tc_layer1(p, d, x, W, b):
    return pl.pallas_call(
        _tc_layer1_body,
        grid=(N // RB,),
        in_specs=[
            pl.BlockSpec((NC, RB, D), lambda i: (0, i, 0)),
            pl.BlockSpec((NC, RB, DEGW), lambda i: (0, i, 0)),
            pl.BlockSpec((RB, D), lambda i: (i, 0)),
            pl.BlockSpec((D, D), lambda i: (0, 0)),
            pl.BlockSpec((1, D), lambda i: (0, 0)),
        ],
        out_specs=pl.BlockSpec((RB, D), lambda i: (i, 0)),
        out_shape=jax.ShapeDtypeStruct((N, D), jnp.float32),
    )(p, d, x, W, b)


def _tc_layer2(p, d, g, W2, b2, Wf, bf, a):
    return pl.pallas_call(
        _tc_layer2_body,
        grid=(N // RB,),
        in_specs=[
            pl.BlockSpec((NC, RB, D), lambda i: (0, i, 0)),
            pl.BlockSpec((NC, RB, DEGW), lambda i: (0, i, 0)),
            pl.BlockSpec((RB, D), lambda i: (i, 0)),
            pl.BlockSpec((D, D), lambda i: (0, 0)),
            pl.BlockSpec((1, D), lambda i: (0, 0)),
            pl.BlockSpec((D, D), lambda i: (0, 0)),
            pl.BlockSpec((1, D), lambda i: (0, 0)),
            pl.BlockSpec((1, D), lambda i: (0, 0)),
        ],
        out_specs=pl.BlockSpec((RB, D), lambda i: (i, 0)),
        out_shape=jax.ShapeDtypeStruct((N, D), jnp.float32),
    )(p, d, g, W2, b2, Wf, bf, a)


def kernel(x, edge_index1, edge_index2, label, idlabel,
           W1, b1, W2, b2, Wfc, bfc, prelu_a):
    del label, idlabel
    zrow = jnp.zeros((RPT, D), jnp.float32)
    zdeg = jnp.zeros((RPT, DEGW), jnp.float32)
    ones = jnp.ones((C, DEGW), jnp.float32)

    ei1 = (edge_index1[0] | (edge_index1[1] << 16)).reshape(E // C, C)
    ei2 = (edge_index2[0] | (edge_index2[1] << 16)).reshape(E // C, C)
    p1, d1 = _sc_aggregate(ei1, x, zrow, zdeg, ones)
    g1 = _tc_layer1(p1, d1, x, W1, b1.reshape(1, D))
    p2, d2 = _sc_aggregate(ei2, g1, zrow, zdeg, ones)
    out = _tc_layer2(p2, d2, g1, W2, b2.reshape(1, D), Wfc, bfc.reshape(1, D),
                     prelu_a.reshape(1, D))
    return out


# speedup vs baseline: 12.1505x; 1.0417x over previous
import functools

import jax
import jax.numpy as jnp
from jax import lax
from jax.experimental import pallas as pl
from jax.experimental.pallas import tpu as pltpu
from jax.experimental.pallas import tpu_sc as plsc

N = 10000
D = 128
E = 320000

NC = 2
NS = 16
NW = NC * NS
EPW = E // NW
C = 80
NCHUNK = EPW // C
NP = 10240
RPT = NP // NS
DEGW = 8

_MESH = plsc.VectorSubcoreMesh(core_axis_name="c", subcore_axis_name="s",
                               num_cores=NC, num_subcores=NS)


@functools.partial(
    pl.kernel,
    out_type=[
        jax.ShapeDtypeStruct((NC, NP, D), jnp.float32),
        jax.ShapeDtypeStruct((NC, NP, DEGW), jnp.float32),
    ],
    mesh=_MESH,
    scratch_types=[
        pltpu.VMEM((NCHUNK, C), jnp.int32),
        pltpu.VMEM((C, D), jnp.float32),
        pltpu.VMEM((C, D), jnp.float32),
        pltpu.VMEM((C, D), jnp.float32),
        pltpu.VMEM((C,), jnp.int32),
        pltpu.VMEM((C,), jnp.int32),
        pltpu.VMEM((C,), jnp.int32),
        pltpu.VMEM((C,), jnp.int32),
        pltpu.VMEM((C,), jnp.int32),
        pltpu.VMEM((C,), jnp.int32),
        pltpu.VMEM((C, DEGW), jnp.float32),
        pltpu.VMEM((C, DEGW), jnp.float32),
        pltpu.SemaphoreType.DMA,
        pltpu.SemaphoreType.DMA,
        pltpu.SemaphoreType.DMA,
        pltpu.SemaphoreType.DMA,
        pltpu.SemaphoreType.DMA,
        pltpu.SemaphoreType.DMA,
        pltpu.SemaphoreType.DMA,
        pltpu.VMEM_SHARED((NP, D), jnp.float32),
        pltpu.VMEM_SHARED((NP, DEGW), jnp.float32),
    ],
    compiler_params=pltpu.CompilerParams(use_tc_tiling_on_sc=False),
)
def _sc_aggregate(ei_hbm, x_hbm,
                  pacc_hbm, pdeg_hbm,
                  eipb, rows0, rows1, rows2, sv0, sv1, sv2, dv0, dv1, dv2,
                  onesv, zdbuf, sg0, sg1, sg2, ss0, ss1, ss2, semd,
                  acc_sh, deg_sh):
    cid = lax.axis_index("c")
    sid = lax.axis_index("s")
    wid = cid * NS + sid
    band = pl.ds(sid * RPT, RPT)
    rows = (rows0, rows1, rows2)
    sv = (sv0, sv1, sv2)
    dv = (dv0, dv1, dv2)
    sg = (sg0, sg1, sg2)
    ss = (ss0, ss1, ss2)

    cbase = wid * NCHUNK
    dei = pltpu.async_copy(ei_hbm.at[pl.ds(cbase, NCHUNK)], eipb, sg0)

    @pl.loop(0, C)
    def _(r):
        for i in range(D // 16):
            rows0[r, pl.ds(i * 16, 16)] = jnp.zeros((16,), jnp.float32)

    @pl.loop(0, C)
    def _(r):
        onesv[r, pl.ds(0, DEGW)] = jnp.ones((DEGW,), jnp.float32)
        zdbuf[r, pl.ds(0, DEGW)] = jnp.zeros((DEGW,), jnp.float32)

    for q in range(RPT // C):
        pltpu.sync_copy(rows0, acc_sh.at[pl.ds(sid * RPT + q * C, C)])
        pltpu.sync_copy(zdbuf, deg_sh.at[pl.ds(sid * RPT + q * C, C)])
    dei.wait()
    plsc.subcore_barrier()

    def unpack(k, b):
        for i in range(C // 16):
            w = eipb[k, pl.ds(i * 16, 16)]
            sv[b][pl.ds(i * 16, 16)] = w & 0xFFFF
            dv[b][pl.ds(i * 16, 16)] = lax.shift_right_logical(w, 16)

    def drain(b):
        pltpu.make_async_copy(rows[b], acc_sh.at[dv[b]], ss[b]).wait()
        pltpu.make_async_copy(onesv, deg_sh.at[dv[b]], semd).wait()

    def gather(k, b):
        pltpu.async_copy(x_hbm.at[sv[b]], rows[b], sg[b])

    def gwait(b):
        pltpu.make_async_copy(x_hbm.at[sv[b]], rows[b], sg[b]).wait()

    def scatter(b):
        pltpu.async_copy(rows[b], acc_sh.at[dv[b]], ss[b], add=True)
        pltpu.async_copy(onesv, deg_sh.at[dv[b]], semd, add=True)

    @pl.loop(0, NCHUNK // 3)
    def _(j):
        k0 = 3 * j
        for b in range(3):
            @pl.when(j > 0)
            def _(b=b):
                drain(b)
            unpack(k0 + b, b)
            gather(k0 + b, b)
        for b in range(3):
            gwait(b)
            scatter(b)

    t0 = 3 * (NCHUNK // 3)
    for b in range(3):
        drain(b)
    for i, k in enumerate(range(t0, NCHUNK)):
        unpack(k, i)
        gather(k, i)
    for i in range(NCHUNK - t0):
        gwait(i)
        scatter(i)
    for i in range(NCHUNK - t0):
        drain(i)

    plsc.subcore_barrier()

    dw0 = pltpu.async_copy(acc_sh.at[band], pacc_hbm.at[cid, band], ss0)
    dw1 = pltpu.async_copy(deg_sh.at[band], pdeg_hbm.at[cid, band], ss1)
    dw0.wait()
    dw1.wait()


RB = 2000


def _tc_layer1_body(p_ref, d_ref, x_ref, w_ref, b_ref, o_ref):
    p = p_ref[0] + p_ref[1]
    deg = (jnp.sum(d_ref[0], axis=1, keepdims=True)
           + jnp.sum(d_ref[1], axis=1, keepdims=True)) * (1.0 / DEGW)
    h = (p + x_ref[...]) / (deg + 1.0)
    g = jnp.dot(h, w_ref[...], preferred_element_type=jnp.float32) + b_ref[...]
    g = jnp.maximum(g, 0.0)
    norm = jnp.maximum(jnp.sqrt(jnp.sum(g * g, axis=1, keepdims=True)), 1e-12)
    o_ref[...] = g / norm


def _tc_layer2_body(p_ref, d_ref, g_ref, w2_ref, b2_ref, wf_ref, bf_ref,
                    a_ref, o_ref):
    p = p_ref[0] + p_ref[1]
    deg = (jnp.sum(d_ref[0], axis=1, keepdims=True)
           + jnp.sum(d_ref[1], axis=1, keepdims=True)) * (1.0 / DEGW)
    h = (p + g_ref[...]) / (deg + 1.0)
    t = jnp.dot(h, w2_ref[...], preferred_element_type=jnp.float32) + b2_ref[...]
    t = jnp.maximum(t, 0.0)
    f = jnp.dot(t, wf_ref[...], preferred_element_type=jnp.float32) + bf_ref[...]
    f = jnp.where(f > 0, f, a_ref[...] * f)
    norm = jnp.maximum(jnp.sqrt(jnp.sum(f * f, axis=1, keepdims=True)), 1e-12)
    o_ref[...] = f / norm


def _tc_layer1(p, d, x, W, b):
    return pl.pallas_call(
        _tc_layer1_body,
        grid=(N // RB,),
        in_specs=[
            pl.BlockSpec((NC, RB, D), lambda i: (0, i, 0)),
            pl.BlockSpec((NC, RB, DEGW), lambda i: (0, i, 0)),
            pl.BlockSpec((RB, D), lambda i: (i, 0)),
            pl.BlockSpec((D, D), lambda i: (0, 0)),
            pl.BlockSpec((1, D), lambda i: (0, 0)),
        ],
        out_specs=pl.BlockSpec((RB, D), lambda i: (i, 0)),
        out_shape=jax.ShapeDtypeStruct((N, D), jnp.float32),
    )(p, d, x, W, b)


def _tc_layer2(p, d, g, W2, b2, Wf, bf, a):
    return pl.pallas_call(
        _tc_layer2_body,
        grid=(N // RB,),
        in_specs=[
            pl.BlockSpec((NC, RB, D), lambda i: (0, i, 0)),
            pl.BlockSpec((NC, RB, DEGW), lambda i: (0, i, 0)),
            pl.BlockSpec((RB, D), lambda i: (i, 0)),
            pl.BlockSpec((D, D), lambda i: (0, 0)),
            pl.BlockSpec((1, D), lambda i: (0, 0)),
            pl.BlockSpec((D, D), lambda i: (0, 0)),
            pl.BlockSpec((1, D), lambda i: (0, 0)),
            pl.BlockSpec((1, D), lambda i: (0, 0)),
        ],
        out_specs=pl.BlockSpec((RB, D), lambda i: (i, 0)),
        out_shape=jax.ShapeDtypeStruct((N, D), jnp.float32),
    )(p, d, g, W2, b2, Wf, bf, a)


def kernel(x, edge_index1, edge_index2, label, idlabel,
           W1, b1, W2, b2, Wfc, bfc, prelu_a):
    del label, idlabel
    ei1 = (edge_index1[0] | (edge_index1[1] << 16)).reshape(E // C, C)
    ei2 = (edge_index2[0] | (edge_index2[1] << 16)).reshape(E // C, C)
    p1, d1 = _sc_aggregate(ei1, x)
    g1 = _tc_layer1(p1, d1, x, W1, b1.reshape(1, D))
    p2, d2 = _sc_aggregate(ei2, g1)
    out = _tc_layer2(p2, d2, g1, W2, b2.reshape(1, D), Wfc, bfc.reshape(1, D),
                     prelu_a.reshape(1, D))
    return out
